# Initial kernel scaffold; baseline (speedup 1.0000x reference)
import jax
import jax.numpy as jnp
from jax.experimental import pallas as pl


def kernel(x, edge_index, W1, b1, W2, b2, W3, b3):
    raise NotImplementedError("write your pallas kernel here")


# speedup vs baseline: 10.9013x; 10.9013x over previous
import functools

import jax
import jax.numpy as jnp
from jax import lax
from jax.experimental import pallas as pl
from jax.experimental.pallas import tpu as pltpu
from jax.experimental.pallas import tpu_sc as plsc

_N = 10000
_E = 320000
_A = 10240
_RPT = _A // 16
_C = 128
_NW = 32
_K = -(-_E // (_NW * _C))
_EPW = _K * _C
_EPAD = _EPW * _NW

_BR = 256
_G = _A // _BR


def _mesh():
    return plsc.VectorSubcoreMesh(core_axis_name="c", subcore_axis_name="s")


def _sc_segsum(D, table, srcp, dstp, zeros):
    @functools.partial(
        pl.kernel,
        out_type=jax.ShapeDtypeStruct((2, _A, D), jnp.float32),
        mesh=_mesh(),
        scratch_types=[
            pltpu.VMEM((_C,), jnp.int32),
            pltpu.VMEM((_C,), jnp.int32),
            pltpu.VMEM((_C, D), jnp.float32),
            pltpu.VMEM_SHARED((_A, D), jnp.float32),
            pltpu.SemaphoreType.DMA,
        ],
        compiler_params=pltpu.CompilerParams(use_tc_tiling_on_sc=False),
    )
    def seg(table_h, src_h, dst_h, zeros_h, out_h, src_v, dst_v, rows_v,
            accum, sem):
        cid = lax.axis_index("c")
        sid = lax.axis_index("s")
        wid = sid * 2 + cid
        r0 = sid * _RPT
        pltpu.sync_copy(zeros_h, accum.at[pl.ds(r0, _RPT)])
        plsc.subcore_barrier()
        base0 = wid * _EPW

        @pl.loop(0, _K)
        def _(j):
            base = base0 + j * _C
            pltpu.sync_copy(src_h.at[pl.ds(base, _C)], src_v)
            pltpu.sync_copy(dst_h.at[pl.ds(base, _C)], dst_v)
            pltpu.async_copy(table_h.at[src_v], rows_v, sem).wait()
            pltpu.sync_copy(rows_v, accum.at[dst_v], add=True)

        plsc.subcore_barrier()
        pltpu.sync_copy(accum.at[pl.ds(r0, _RPT)],
                        out_h.at[cid, pl.ds(r0, _RPT)])

    return seg(table, srcp, dstp, zeros)


def _tc_matmul(x, W):
    Din, Dout = W.shape

    def body(x_ref, w_ref, o_ref):
        o_ref[...] = jnp.dot(x_ref[...], w_ref[...],
                             preferred_element_type=jnp.float32)

    return pl.pallas_call(
        body,
        grid=(_G,),
        in_specs=[pl.BlockSpec((_BR, Din), lambda i: (i, 0)),
                  pl.BlockSpec((Din, Dout), lambda i: (0, 0))],
        out_specs=pl.BlockSpec((_BR, Dout), lambda i: (i, 0)),
        out_shape=jax.ShapeDtypeStruct((_A, Dout), jnp.float32),
    )(x, W)


def _tc_scale(degT, p1):
    D = p1.shape[1]

    def body(deg_ref, p_ref, s_ref, ps_ref):
        deg = deg_ref[...]
        s = lax.rsqrt(deg[:, 0:1] + deg[:, 1:2] + 1.0)
        s_ref[...] = s
        ps_ref[...] = p_ref[...] * s

    return pl.pallas_call(
        body,
        grid=(_G,),
        in_specs=[pl.BlockSpec((_BR, 2), lambda i: (i, 0)),
                  pl.BlockSpec((_BR, D), lambda i: (i, 0))],
        out_specs=[pl.BlockSpec((_BR, 1), lambda i: (i, 0)),
                   pl.BlockSpec((_BR, D), lambda i: (i, 0))],
        out_shape=[jax.ShapeDtypeStruct((_A, 1), jnp.float32),
                   jax.ShapeDtypeStruct((_A, D), jnp.float32)],
    )(degT, p1)


def _tc_layer(agg, p, s, b, W):
    Din = p.shape[1]
    Dout = W.shape[1]

    def body(a_ref, p_ref, s_ref, b_ref, w_ref, q_ref, qs_ref):
        s1 = s_ref[...]
        pb = p_ref[...]
        h = s1 * (a_ref[0] + a_ref[1]) + (s1 * s1) * pb + b_ref[...]
        h = jnp.maximum(h, 0.0)
        q = jnp.dot(h, w_ref[...], preferred_element_type=jnp.float32)
        q_ref[...] = q
        qs_ref[...] = q * s1

    return pl.pallas_call(
        body,
        grid=(_G,),
        in_specs=[pl.BlockSpec((2, _BR, Din), lambda i: (0, i, 0)),
                  pl.BlockSpec((_BR, Din), lambda i: (i, 0)),
                  pl.BlockSpec((_BR, 1), lambda i: (i, 0)),
                  pl.BlockSpec((1, Din), lambda i: (0, 0)),
                  pl.BlockSpec((Din, Dout), lambda i: (0, 0))],
        out_specs=[pl.BlockSpec((_BR, Dout), lambda i: (i, 0)),
                   pl.BlockSpec((_BR, Dout), lambda i: (i, 0))],
        out_shape=[jax.ShapeDtypeStruct((_A, Dout), jnp.float32),
                   jax.ShapeDtypeStruct((_A, Dout), jnp.float32)],
    )(agg, p, s, b, W)


def _tc_final(agg, p, s, b):
    D = p.shape[1]

    def body(a_ref, p_ref, s_ref, b_ref, o_ref):
        s1 = s_ref[...]
        o = s1 * (a_ref[0] + a_ref[1]) + (s1 * s1) * p_ref[...] + b_ref[...]
        o2 = o[:, 0:2]
        m = jnp.max(o2, axis=1, keepdims=True)
        lse = m + jnp.log(jnp.sum(jnp.exp(o2 - m), axis=1, keepdims=True))
        o_ref[...] = o - lse

    return pl.pallas_call(
        body,
        grid=(_G,),
        in_specs=[pl.BlockSpec((2, _BR, D), lambda i: (0, i, 0)),
                  pl.BlockSpec((_BR, D), lambda i: (i, 0)),
                  pl.BlockSpec((_BR, 1), lambda i: (i, 0)),
                  pl.BlockSpec((1, D), lambda i: (0, 0))],
        out_specs=pl.BlockSpec((_BR, D), lambda i: (i, 0)),
        out_shape=jax.ShapeDtypeStruct((_A, D), jnp.float32),
    )(agg, p, s, b)


def kernel(x, edge_index, W1, b1, W2, b2, W3, b3):
    src = edge_index[0].astype(jnp.int32)
    dst = edge_index[1].astype(jnp.int32)
    pad = jnp.full((_EPAD - _E,), _N, dtype=jnp.int32)
    srcp = jnp.concatenate([src, pad])
    dstp = jnp.concatenate([dst, pad])

    x_pad = jnp.zeros((_A, x.shape[1]), x.dtype).at[:_N].set(x)
    zeros64 = jnp.zeros((_RPT, 64), jnp.float32)
    zeros16 = jnp.zeros((_RPT, 16), jnp.float32)
    deg_table = jnp.zeros((_A, 16), jnp.float32).at[:_N].set(1.0)
    W3p = jnp.zeros((W3.shape[0], 16), jnp.float32).at[:, :2].set(W3)
    b1r = b1.reshape(1, -1)
    b2r = b2.reshape(1, -1)
    b3r = jnp.zeros((1, 16), jnp.float32).at[0, :2].set(b3)

    deg2 = _sc_segsum(16, deg_table, dstp, dstp, zeros16)
    p1 = _tc_matmul(x_pad, W1)

    degT = deg2[:, :, 0].T
    s, ps1 = _tc_scale(degT, p1)

    agg1 = _sc_segsum(64, ps1, srcp, dstp, zeros64)
    p2, ps2 = _tc_layer(agg1, p1, s, b1r, W2)

    agg2 = _sc_segsum(64, ps2, srcp, dstp, zeros64)
    p3, ps3 = _tc_layer(agg2, p2, s, b2r, W3p)

    agg3 = _sc_segsum(16, ps3, srcp, dstp, zeros16)
    out16 = _tc_final(agg3, p3, s, b3r)

    return out16[:_N, :2]


# speedup vs baseline: 14.3734x; 1.3185x over previous
import functools

import jax
import jax.numpy as jnp
from jax import lax
from jax.experimental import pallas as pl
from jax.experimental.pallas import tpu as pltpu
from jax.experimental.pallas import tpu_sc as plsc

_N = 10000
_E = 320000
_A = 10240
_RPT = _A // 16
_C = 128
_NW = 32
_K = 80
_EPW = _K * _C
_EPAD = _EPW * _NW

_BR = 256
_G = _A // _BR


def _mesh():
    return plsc.VectorSubcoreMesh(core_axis_name="c", subcore_axis_name="s")


def _sc_segsum(D, table, srcp, dstp, zeros):
    @functools.partial(
        pl.kernel,
        out_type=jax.ShapeDtypeStruct((2, _A, D), jnp.float32),
        mesh=_mesh(),
        scratch_types=[
            pltpu.VMEM((_K, _C), jnp.int32),
            pltpu.VMEM((_K, _C), jnp.int32),
            pltpu.VMEM((4, _C, D), jnp.float32),
            pltpu.VMEM_SHARED((_A, D), jnp.float32),
            [pltpu.SemaphoreType.DMA] * 4,
            [pltpu.SemaphoreType.DMA] * 4,
        ],
        compiler_params=pltpu.CompilerParams(use_tc_tiling_on_sc=False),
    )
    def seg(table_h, src_h, dst_h, zeros_h, out_h, src_v, dst_v, rows_v,
            accum, sg, ss):
        cid = lax.axis_index("c")
        sid = lax.axis_index("s")
        wid = sid * 2 + cid
        r0 = sid * _RPT
        base0 = wid * _K
        pltpu.sync_copy(src_h.at[pl.ds(base0, _K)], src_v)
        pltpu.sync_copy(dst_h.at[pl.ds(base0, _K)], dst_v)
        pltpu.sync_copy(zeros_h, accum.at[pl.ds(r0, _RPT)])
        plsc.subcore_barrier()

        def gather(j, u):
            pltpu.async_copy(table_h.at[src_v.at[j]], rows_v.at[u], sg[u])

        def wait_gather(u):
            pltpu.make_async_copy(table_h.at[src_v.at[0]], rows_v.at[u],
                                  sg[u]).wait()

        def scatter(j, u):
            pltpu.async_copy(rows_v.at[u], accum.at[dst_v.at[j]], ss[u],
                             add=True)

        def wait_scatter(u):
            pltpu.make_async_copy(rows_v.at[u], accum.at[dst_v.at[0]],
                                  ss[u]).wait()

        gather(0, 0)
        gather(1, 1)
        for j in range(4):
            u = j % 4
            wait_gather(u)
            scatter(j, u)
            if j >= 2:
                wait_scatter((u + 2) % 4)
            gather(j + 2, (u + 2) % 4)

        @pl.loop(4, _K - 4, step=4)
        def _(jb):
            for u in range(4):
                j = jb + u
                wait_gather(u)
                scatter(j, u)
                wait_scatter((u + 2) % 4)
                gather(j + 2, (u + 2) % 4)

        for u in range(4):
            j = _K - 4 + u
            wait_gather(u)
            scatter(j, u)
            if u < 2:
                wait_scatter((u + 2) % 4)
                gather(j + 2, (u + 2) % 4)
        for u in range(4):
            wait_scatter(u)

        plsc.subcore_barrier()
        pltpu.sync_copy(accum.at[pl.ds(r0, _RPT)],
                        out_h.at[cid, pl.ds(r0, _RPT)])

    return seg(table, srcp, dstp, zeros)


def _sc_deg(dstp2, ones, zeros):
    _W = 8

    @functools.partial(
        pl.kernel,
        out_type=jax.ShapeDtypeStruct((2, _A, 16), jnp.float32),
        mesh=_mesh(),
        scratch_types=[
            pltpu.VMEM((_K, _C), jnp.int32),
            pltpu.VMEM((_C, 16), jnp.float32),
            pltpu.VMEM_SHARED((_A, 16), jnp.float32),
            pltpu.SemaphoreType.DMA,
        ],
        compiler_params=pltpu.CompilerParams(use_tc_tiling_on_sc=False),
    )
    def deg(dst_h, ones_h, zeros_h, out_h, dst_v, ones_v, accum, sem):
        cid = lax.axis_index("c")
        sid = lax.axis_index("s")
        wid = sid * 2 + cid
        r0 = sid * _RPT
        pltpu.sync_copy(ones_h, ones_v)
        pltpu.sync_copy(dst_h.at[pl.ds(wid * _K, _K)], dst_v)
        pltpu.sync_copy(zeros_h, accum.at[pl.ds(r0, _RPT)])
        plsc.subcore_barrier()

        def wait_one():
            pltpu.make_async_copy(ones_v, accum.at[dst_v.at[0]], sem).wait()

        for j in range(_W):
            pltpu.async_copy(ones_v, accum.at[dst_v.at[j]], sem, add=True)

        @pl.loop(_W, _K)
        def _(j):
            wait_one()
            pltpu.async_copy(ones_v, accum.at[dst_v.at[j]], sem, add=True)

        for _ in range(_W):
            wait_one()

        plsc.subcore_barrier()
        pltpu.sync_copy(accum.at[pl.ds(r0, _RPT)],
                        out_h.at[cid, pl.ds(r0, _RPT)])

    return deg(dstp2, ones, zeros)


def _tc_matmul(x, W):
    Din, Dout = W.shape

    def body(x_ref, w_ref, o_ref):
        o_ref[...] = jnp.dot(x_ref[...], w_ref[...],
                             preferred_element_type=jnp.float32)

    return pl.pallas_call(
        body,
        grid=(_G,),
        in_specs=[pl.BlockSpec((_BR, Din), lambda i: (i, 0)),
                  pl.BlockSpec((Din, Dout), lambda i: (0, 0))],
        out_specs=pl.BlockSpec((_BR, Dout), lambda i: (i, 0)),
        out_shape=jax.ShapeDtypeStruct((_A, Dout), jnp.float32),
    )(x, W)


def _tc_scale(degT, p1):
    D = p1.shape[1]

    def body(deg_ref, p_ref, s_ref, ps_ref):
        deg = deg_ref[...]
        s = lax.rsqrt(deg[:, 0:1] + deg[:, 1:2] + 1.0)
        s_ref[...] = s
        ps_ref[...] = p_ref[...] * s

    return pl.pallas_call(
        body,
        grid=(_G,),
        in_specs=[pl.BlockSpec((_BR, 2), lambda i: (i, 0)),
                  pl.BlockSpec((_BR, D), lambda i: (i, 0))],
        out_specs=[pl.BlockSpec((_BR, 1), lambda i: (i, 0)),
                   pl.BlockSpec((_BR, D), lambda i: (i, 0))],
        out_shape=[jax.ShapeDtypeStruct((_A, 1), jnp.float32),
                   jax.ShapeDtypeStruct((_A, D), jnp.float32)],
    )(degT, p1)


def _tc_layer(agg, p, s, b, W):
    Din = p.shape[1]
    Dout = W.shape[1]

    def body(a_ref, p_ref, s_ref, b_ref, w_ref, q_ref, qs_ref):
        s1 = s_ref[...]
        pb = p_ref[...]
        h = s1 * (a_ref[0] + a_ref[1]) + (s1 * s1) * pb + b_ref[...]
        h = jnp.maximum(h, 0.0)
        q = jnp.dot(h, w_ref[...], preferred_element_type=jnp.float32)
        q_ref[...] = q
        qs_ref[...] = q * s1

    return pl.pallas_call(
        body,
        grid=(_G,),
        in_specs=[pl.BlockSpec((2, _BR, Din), lambda i: (0, i, 0)),
                  pl.BlockSpec((_BR, Din), lambda i: (i, 0)),
                  pl.BlockSpec((_BR, 1), lambda i: (i, 0)),
                  pl.BlockSpec((1, Din), lambda i: (0, 0)),
                  pl.BlockSpec((Din, Dout), lambda i: (0, 0))],
        out_specs=[pl.BlockSpec((_BR, Dout), lambda i: (i, 0)),
                   pl.BlockSpec((_BR, Dout), lambda i: (i, 0))],
        out_shape=[jax.ShapeDtypeStruct((_A, Dout), jnp.float32),
                   jax.ShapeDtypeStruct((_A, Dout), jnp.float32)],
    )(agg, p, s, b, W)


def _tc_final(agg, p, s, b):
    D = p.shape[1]

    def body(a_ref, p_ref, s_ref, b_ref, o_ref):
        s1 = s_ref[...]
        o = s1 * (a_ref[0] + a_ref[1]) + (s1 * s1) * p_ref[...] + b_ref[...]
        o2 = o[:, 0:2]
        m = jnp.max(o2, axis=1, keepdims=True)
        lse = m + jnp.log(jnp.sum(jnp.exp(o2 - m), axis=1, keepdims=True))
        o_ref[...] = o - lse

    return pl.pallas_call(
        body,
        grid=(_G,),
        in_specs=[pl.BlockSpec((2, _BR, D), lambda i: (0, i, 0)),
                  pl.BlockSpec((_BR, D), lambda i: (i, 0)),
                  pl.BlockSpec((_BR, 1), lambda i: (i, 0)),
                  pl.BlockSpec((1, D), lambda i: (0, 0))],
        out_specs=pl.BlockSpec((_BR, D), lambda i: (i, 0)),
        out_shape=jax.ShapeDtypeStruct((_A, D), jnp.float32),
    )(agg, p, s, b)


def kernel(x, edge_index, W1, b1, W2, b2, W3, b3):
    src = edge_index[0].astype(jnp.int32)
    dst = edge_index[1].astype(jnp.int32)
    pad = jnp.full((_EPAD - _E,), _N, dtype=jnp.int32)
    srcp = jnp.concatenate([src, pad]).reshape(_NW * _K, _C)
    dstp = jnp.concatenate([dst, pad]).reshape(_NW * _K, _C)

    x_pad = jnp.zeros((_A, x.shape[1]), x.dtype).at[:_N].set(x)
    zeros64 = jnp.zeros((_RPT, 64), jnp.float32)
    zeros16 = jnp.zeros((_RPT, 16), jnp.float32)
    ones16 = jnp.ones((_C, 16), jnp.float32)
    W3p = jnp.zeros((W3.shape[0], 16), jnp.float32).at[:, :2].set(W3)
    b1r = b1.reshape(1, -1)
    b2r = b2.reshape(1, -1)
    b3r = jnp.zeros((1, 16), jnp.float32).at[0, :2].set(b3)

    deg2 = _sc_deg(dstp, ones16, zeros16)
    p1 = _tc_matmul(x_pad, W1)

    degT = deg2[:, :, 0].T
    s, ps1 = _tc_scale(degT, p1)

    agg1 = _sc_segsum(64, ps1, srcp, dstp, zeros64)
    p2, ps2 = _tc_layer(agg1, p1, s, b1r, W2)

    agg2 = _sc_segsum(64, ps2, srcp, dstp, zeros64)
    p3, ps3 = _tc_layer(agg2, p2, s, b2r, W3p)

    agg3 = _sc_segsum(16, ps3, srcp, dstp, zeros16)
    out16 = _tc_final(agg3, p3, s, b3r)

    return out16[:_N, :2]


# speedup vs baseline: 33.7615x; 2.3489x over previous
import functools

import jax
import jax.numpy as jnp
from jax import lax
from jax.experimental import pallas as pl
from jax.experimental.pallas import tpu as pltpu
from jax.experimental.pallas import tpu_sc as plsc

_N = 10000
_E = 320000
_A = 10240
_RPT = _A // 16
_C = 128
_NW = 32
_K = 80
_EPW = _K * _C
_EPAD = _EPW * _NW

_BR = 512
_G = _A // _BR


def _mesh():
    return plsc.VectorSubcoreMesh(core_axis_name="c", subcore_axis_name="s")


def _sc_segsum(D, table, srcp, dstp, zeros):
    @functools.partial(
        pl.kernel,
        out_type=jax.ShapeDtypeStruct((2, _A, D), jnp.float32),
        mesh=_mesh(),
        scratch_types=[
            pltpu.VMEM((_K, _C), jnp.int32),
            pltpu.VMEM((_K, _C), jnp.int32),
            pltpu.VMEM((4, _C, D), jnp.float32),
            pltpu.VMEM_SHARED((_A, D), jnp.float32),
            [pltpu.SemaphoreType.DMA] * 4,
            [pltpu.SemaphoreType.DMA] * 4,
        ],
        compiler_params=pltpu.CompilerParams(use_tc_tiling_on_sc=False),
    )
    def seg(table_h, src_h, dst_h, zeros_h, out_h, src_v, dst_v, rows_v,
            accum, sg, ss):
        cid = lax.axis_index("c")
        sid = lax.axis_index("s")
        wid = sid * 2 + cid
        r0 = sid * _RPT
        base0 = wid * _K
        pltpu.sync_copy(src_h.at[pl.ds(base0, _K)], src_v)
        pltpu.sync_copy(dst_h.at[pl.ds(base0, _K)], dst_v)
        pltpu.sync_copy(zeros_h, accum.at[pl.ds(r0, _RPT)])
        plsc.subcore_barrier()

        def gather(j, u):
            pltpu.async_copy(table_h.at[src_v.at[j]], rows_v.at[u], sg[u])

        def wait_gather(u):
            pltpu.make_async_copy(table_h.at[src_v.at[0]], rows_v.at[u],
                                  sg[u]).wait()

        def scatter(j, u):
            pltpu.async_copy(rows_v.at[u], accum.at[dst_v.at[j]], ss[u],
                             add=True)

        def wait_scatter(u):
            pltpu.make_async_copy(rows_v.at[u], accum.at[dst_v.at[0]],
                                  ss[u]).wait()

        gather(0, 0)
        gather(1, 1)
        for j in range(4):
            u = j % 4
            wait_gather(u)
            scatter(j, u)
            if j >= 2:
                wait_scatter((u + 2) % 4)
            gather(j + 2, (u + 2) % 4)

        @pl.loop(4, _K - 4, step=4)
        def _(jb):
            for u in range(4):
                j = jb + u
                wait_gather(u)
                scatter(j, u)
                wait_scatter((u + 2) % 4)
                gather(j + 2, (u + 2) % 4)

        for u in range(4):
            j = _K - 4 + u
            wait_gather(u)
            scatter(j, u)
            if u < 2:
                wait_scatter((u + 2) % 4)
                gather(j + 2, (u + 2) % 4)
        for u in range(4):
            wait_scatter(u)

        plsc.subcore_barrier()
        pltpu.sync_copy(accum.at[pl.ds(r0, _RPT)],
                        out_h.at[cid, pl.ds(r0, _RPT)])

    return seg(table, srcp, dstp, zeros)


def _sc_deg(dstp2, ones, zeros):
    _W = 8

    @functools.partial(
        pl.kernel,
        out_type=jax.ShapeDtypeStruct((2, _A, 16), jnp.float32),
        mesh=_mesh(),
        scratch_types=[
            pltpu.VMEM((_K, _C), jnp.int32),
            pltpu.VMEM((_C, 16), jnp.float32),
            pltpu.VMEM_SHARED((_A, 16), jnp.float32),
            pltpu.SemaphoreType.DMA,
        ],
        compiler_params=pltpu.CompilerParams(use_tc_tiling_on_sc=False),
    )
    def deg(dst_h, ones_h, zeros_h, out_h, dst_v, ones_v, accum, sem):
        cid = lax.axis_index("c")
        sid = lax.axis_index("s")
        wid = sid * 2 + cid
        r0 = sid * _RPT
        pltpu.sync_copy(ones_h, ones_v)
        pltpu.sync_copy(dst_h.at[pl.ds(wid * _K, _K)], dst_v)
        pltpu.sync_copy(zeros_h, accum.at[pl.ds(r0, _RPT)])
        plsc.subcore_barrier()

        def wait_one():
            pltpu.make_async_copy(ones_v, accum.at[dst_v.at[0]], sem).wait()

        for j in range(_W):
            pltpu.async_copy(ones_v, accum.at[dst_v.at[j]], sem, add=True)

        @pl.loop(_W, _K)
        def _(j):
            wait_one()
            pltpu.async_copy(ones_v, accum.at[dst_v.at[j]], sem, add=True)

        for _ in range(_W):
            wait_one()

        plsc.subcore_barrier()
        pltpu.sync_copy(accum.at[pl.ds(r0, _RPT)],
                        out_h.at[cid, pl.ds(r0, _RPT)])

    return deg(dstp2, ones, zeros)


def _row_keep(i, rows):
    ids = i * _BR + lax.broadcasted_iota(jnp.int32, (rows, 1), 0)
    return ids < _N


def _tc_prep(x, W, deg2):
    Din, Dout = W.shape

    def body(x_ref, w_ref, d_ref, p_ref, s_ref, ps_ref):
        i = pl.program_id(0)
        p = jnp.dot(x_ref[...], w_ref[...], preferred_element_type=jnp.float32)
        s = lax.rsqrt(d_ref[0, :, 0:1] + d_ref[1, :, 0:1] + 1.0)
        p_ref[...] = p
        s_ref[...] = s
        ps_ref[...] = jnp.where(_row_keep(i, _BR), p * s, 0.0)

    return pl.pallas_call(
        body,
        grid=(_G,),
        in_specs=[pl.BlockSpec((_BR, Din), lambda i: (i, 0)),
                  pl.BlockSpec((Din, Dout), lambda i: (0, 0)),
                  pl.BlockSpec((2, _BR, 16), lambda i: (0, i, 0))],
        out_specs=[pl.BlockSpec((_BR, Dout), lambda i: (i, 0)),
                   pl.BlockSpec((_BR, 1), lambda i: (i, 0)),
                   pl.BlockSpec((_BR, Dout), lambda i: (i, 0))],
        out_shape=[jax.ShapeDtypeStruct((_A, Dout), jnp.float32),
                   jax.ShapeDtypeStruct((_A, 1), jnp.float32),
                   jax.ShapeDtypeStruct((_A, Dout), jnp.float32)],
    )(x, W, deg2)


def _tc_layer(agg, p, s, b, W):
    Din = p.shape[1]
    Dout = W.shape[1]

    def body(a_ref, p_ref, s_ref, b_ref, w_ref, q_ref, qs_ref):
        i = pl.program_id(0)
        s1 = s_ref[...]
        pb = p_ref[...]
        h = s1 * (a_ref[0] + a_ref[1]) + (s1 * s1) * pb + b_ref[...]
        h = jnp.maximum(h, 0.0)
        q = jnp.dot(h, w_ref[...], preferred_element_type=jnp.float32)
        q_ref[...] = q
        qs_ref[...] = jnp.where(_row_keep(i, _BR), q * s1, 0.0)

    return pl.pallas_call(
        body,
        grid=(_G,),
        in_specs=[pl.BlockSpec((2, _BR, Din), lambda i: (0, i, 0)),
                  pl.BlockSpec((_BR, Din), lambda i: (i, 0)),
                  pl.BlockSpec((_BR, 1), lambda i: (i, 0)),
                  pl.BlockSpec((1, Din), lambda i: (0, 0)),
                  pl.BlockSpec((Din, Dout), lambda i: (0, 0))],
        out_specs=[pl.BlockSpec((_BR, Dout), lambda i: (i, 0)),
                   pl.BlockSpec((_BR, Dout), lambda i: (i, 0))],
        out_shape=[jax.ShapeDtypeStruct((_A, Dout), jnp.float32),
                   jax.ShapeDtypeStruct((_A, Dout), jnp.float32)],
    )(agg, p, s, b, W)


def _tc_final(agg, p, s, b):
    D = p.shape[1]

    def body(a_ref, p_ref, s_ref, b_ref, o_ref):
        s1 = s_ref[...]
        o = s1 * (a_ref[0] + a_ref[1]) + (s1 * s1) * p_ref[...] + b_ref[...]
        o2 = o[:, 0:2]
        m = jnp.max(o2, axis=1, keepdims=True)
        lse = m + jnp.log(jnp.sum(jnp.exp(o2 - m), axis=1, keepdims=True))
        o_ref[...] = o - lse

    return pl.pallas_call(
        body,
        grid=(_G,),
        in_specs=[pl.BlockSpec((2, _BR, D), lambda i: (0, i, 0)),
                  pl.BlockSpec((_BR, D), lambda i: (i, 0)),
                  pl.BlockSpec((_BR, 1), lambda i: (i, 0)),
                  pl.BlockSpec((1, D), lambda i: (0, 0))],
        out_specs=pl.BlockSpec((_BR, D), lambda i: (i, 0)),
        out_shape=jax.ShapeDtypeStruct((_A, D), jnp.float32),
    )(agg, p, s, b)


def kernel(x, edge_index, W1, b1, W2, b2, W3, b3):
    src = edge_index[0].astype(jnp.int32)
    dst = edge_index[1].astype(jnp.int32)
    pad = _N + (jnp.arange(_EPAD - _E, dtype=jnp.int32) % (_A - _N))
    srcp = jnp.concatenate([src, pad]).reshape(_NW * _K, _C)
    dstp = jnp.concatenate([dst, pad]).reshape(_NW * _K, _C)

    x_pad = jnp.zeros((_A, x.shape[1]), x.dtype).at[:_N].set(x)
    zeros64 = jnp.zeros((_RPT, 64), jnp.float32)
    zeros16 = jnp.zeros((_RPT, 16), jnp.float32)
    ones16 = jnp.ones((_C, 16), jnp.float32)
    W3p = jnp.zeros((W3.shape[0], 16), jnp.float32).at[:, :2].set(W3)
    b1r = b1.reshape(1, -1)
    b2r = b2.reshape(1, -1)
    b3r = jnp.zeros((1, 16), jnp.float32).at[0, :2].set(b3)

    deg2 = _sc_deg(dstp, ones16, zeros16)
    p1, s, ps1 = _tc_prep(x_pad, W1, deg2)

    agg1 = _sc_segsum(64, ps1, srcp, dstp, zeros64)
    p2, ps2 = _tc_layer(agg1, p1, s, b1r, W2)

    agg2 = _sc_segsum(64, ps2, srcp, dstp, zeros64)
    p3, ps3 = _tc_layer(agg2, p2, s, b2r, W3p)

    agg3 = _sc_segsum(16, ps3, srcp, dstp, zeros16)
    out16 = _tc_final(agg3, p3, s, b3r)

    return out16[:_N, :2]


# speedup vs baseline: 39.4337x; 1.1680x over previous
import functools

import jax
import jax.numpy as jnp
from jax import lax
from jax.experimental import pallas as pl
from jax.experimental.pallas import tpu as pltpu
from jax.experimental.pallas import tpu_sc as plsc

_N = 10000
_E = 320000
_A = 10240
_RPT = _A // 16
_C = 128
_NW = 32
_K = 80
_EPW = _K * _C
_EPAD = _EPW * _NW

_BR = 1024
_G = _A // _BR

_NS = 8
_AH = 4


def _mesh():
    return plsc.VectorSubcoreMesh(core_axis_name="c", subcore_axis_name="s")


def _sc_segsum(D, table, srcp, dstp, zeros):
    @functools.partial(
        pl.kernel,
        out_type=jax.ShapeDtypeStruct((2, _A, D), jnp.float32),
        mesh=_mesh(),
        scratch_types=[
            pltpu.VMEM((_K, _C), jnp.int32),
            pltpu.VMEM((_K, _C), jnp.int32),
            pltpu.VMEM((_NS, _C, D), jnp.float32),
            pltpu.VMEM_SHARED((_A, D), jnp.float32),
            [pltpu.SemaphoreType.DMA] * _NS,
            [pltpu.SemaphoreType.DMA] * _NS,
        ],
        compiler_params=pltpu.CompilerParams(use_tc_tiling_on_sc=False),
    )
    def seg(table_h, src_h, dst_h, zeros_h, out_h, src_v, dst_v, rows_v,
            accum, sg, ss):
        cid = lax.axis_index("c")
        sid = lax.axis_index("s")
        wid = sid * 2 + cid
        r0 = sid * _RPT
        base0 = wid * _K
        pltpu.sync_copy(src_h.at[pl.ds(base0, _K)], src_v)
        pltpu.sync_copy(dst_h.at[pl.ds(base0, _K)], dst_v)
        pltpu.sync_copy(zeros_h, accum.at[pl.ds(r0, _RPT)])
        plsc.subcore_barrier()

        def gather(j, u):
            pltpu.async_copy(table_h.at[src_v.at[j]], rows_v.at[u], sg[u])

        def wait_gather(u):
            pltpu.make_async_copy(table_h.at[src_v.at[0]], rows_v.at[u],
                                  sg[u]).wait()

        def scatter(j, u):
            pltpu.async_copy(rows_v.at[u], accum.at[dst_v.at[j]], ss[u],
                             add=True)

        def wait_scatter(u):
            pltpu.make_async_copy(rows_v.at[u], accum.at[dst_v.at[0]],
                                  ss[u]).wait()

        for j in range(_AH):
            gather(j, j)
        for j in range(_NS):
            u = j % _NS
            wait_gather(u)
            scatter(j, u)
            if j >= _AH:
                wait_scatter((u + _AH) % _NS)
            gather(j + _AH, (u + _AH) % _NS)

        @pl.loop(_NS, _K - _NS, step=_NS)
        def _(jb):
            for u in range(_NS):
                j = jb + u
                wait_gather(u)
                scatter(j, u)
                wait_scatter((u + _AH) % _NS)
                gather(j + _AH, (u + _AH) % _NS)

        for u in range(_NS):
            j = _K - _NS + u
            wait_gather(u)
            scatter(j, u)
            if u < _NS - _AH:
                wait_scatter((u + _AH) % _NS)
                gather(j + _AH, (u + _AH) % _NS)
        for u in range(_NS):
            wait_scatter(u)

        plsc.subcore_barrier()
        pltpu.sync_copy(accum.at[pl.ds(r0, _RPT)],
                        out_h.at[cid, pl.ds(r0, _RPT)])

    return seg(table, srcp, dstp, zeros)


def _sc_deg(dstp2, ones, zeros):
    _W = 8

    @functools.partial(
        pl.kernel,
        out_type=jax.ShapeDtypeStruct((2, _A, 16), jnp.float32),
        mesh=_mesh(),
        scratch_types=[
            pltpu.VMEM((_K, _C), jnp.int32),
            pltpu.VMEM((_C, 16), jnp.float32),
            pltpu.VMEM_SHARED((_A, 16), jnp.float32),
            pltpu.SemaphoreType.DMA,
        ],
        compiler_params=pltpu.CompilerParams(use_tc_tiling_on_sc=False),
    )
    def deg(dst_h, ones_h, zeros_h, out_h, dst_v, ones_v, accum, sem):
        cid = lax.axis_index("c")
        sid = lax.axis_index("s")
        wid = sid * 2 + cid
        r0 = sid * _RPT
        pltpu.sync_copy(ones_h, ones_v)
        pltpu.sync_copy(dst_h.at[pl.ds(wid * _K, _K)], dst_v)
        pltpu.sync_copy(zeros_h, accum.at[pl.ds(r0, _RPT)])
        plsc.subcore_barrier()

        def wait_one():
            pltpu.make_async_copy(ones_v, accum.at[dst_v.at[0]], sem).wait()

        for j in range(_W):
            pltpu.async_copy(ones_v, accum.at[dst_v.at[j]], sem, add=True)

        @pl.loop(_W, _K)
        def _(j):
            wait_one()
            pltpu.async_copy(ones_v, accum.at[dst_v.at[j]], sem, add=True)

        for _ in range(_W):
            wait_one()

        plsc.subcore_barrier()
        pltpu.sync_copy(accum.at[pl.ds(r0, _RPT)],
                        out_h.at[cid, pl.ds(r0, _RPT)])

    return deg(dstp2, ones, zeros)


def _row_keep(i, rows):
    ids = i * _BR + lax.broadcasted_iota(jnp.int32, (rows, 1), 0)
    return ids < _N


def _tc_prep(x, W, deg2):
    Din, Dout = W.shape

    def body(x_ref, w_ref, d_ref, p_ref, s_ref, ps_ref):
        i = pl.program_id(0)
        p = jnp.dot(x_ref[...], w_ref[...], preferred_element_type=jnp.float32)
        s = lax.rsqrt(d_ref[0, :, 0:1] + d_ref[1, :, 0:1] + 1.0)
        p_ref[...] = p
        s_ref[...] = s
        ps_ref[...] = jnp.where(_row_keep(i, _BR), p * s, 0.0)

    return pl.pallas_call(
        body,
        grid=(_G,),
        in_specs=[pl.BlockSpec((_BR, Din), lambda i: (i, 0)),
                  pl.BlockSpec((Din, Dout), lambda i: (0, 0)),
                  pl.BlockSpec((2, _BR, 16), lambda i: (0, i, 0))],
        out_specs=[pl.BlockSpec((_BR, Dout), lambda i: (i, 0)),
                   pl.BlockSpec((_BR, 1), lambda i: (i, 0)),
                   pl.BlockSpec((_BR, Dout), lambda i: (i, 0))],
        out_shape=[jax.ShapeDtypeStruct((_A, Dout), jnp.float32),
                   jax.ShapeDtypeStruct((_A, 1), jnp.float32),
                   jax.ShapeDtypeStruct((_A, Dout), jnp.float32)],
    )(x, W, deg2)


def _tc_layer(agg, p, s, b, W):
    Din = p.shape[1]
    Dout = W.shape[1]

    def body(a_ref, p_ref, s_ref, b_ref, w_ref, q_ref, qs_ref):
        i = pl.program_id(0)
        s1 = s_ref[...]
        pb = p_ref[...]
        h = s1 * (a_ref[0] + a_ref[1]) + (s1 * s1) * pb + b_ref[...]
        h = jnp.maximum(h, 0.0)
        q = jnp.dot(h, w_ref[...], preferred_element_type=jnp.float32)
        q_ref[...] = q
        qs_ref[...] = jnp.where(_row_keep(i, _BR), q * s1, 0.0)

    return pl.pallas_call(
        body,
        grid=(_G,),
        in_specs=[pl.BlockSpec((2, _BR, Din), lambda i: (0, i, 0)),
                  pl.BlockSpec((_BR, Din), lambda i: (i, 0)),
                  pl.BlockSpec((_BR, 1), lambda i: (i, 0)),
                  pl.BlockSpec((1, Din), lambda i: (0, 0)),
                  pl.BlockSpec((Din, Dout), lambda i: (0, 0))],
        out_specs=[pl.BlockSpec((_BR, Dout), lambda i: (i, 0)),
                   pl.BlockSpec((_BR, Dout), lambda i: (i, 0))],
        out_shape=[jax.ShapeDtypeStruct((_A, Dout), jnp.float32),
                   jax.ShapeDtypeStruct((_A, Dout), jnp.float32)],
    )(agg, p, s, b, W)


def _tc_final(agg, p, s, b):
    D = p.shape[1]

    def body(a_ref, p_ref, s_ref, b_ref, o_ref):
        s1 = s_ref[...]
        o = s1 * (a_ref[0] + a_ref[1]) + (s1 * s1) * p_ref[...] + b_ref[...]
        o2 = o[:, 0:2]
        m = jnp.max(o2, axis=1, keepdims=True)
        lse = m + jnp.log(jnp.sum(jnp.exp(o2 - m), axis=1, keepdims=True))
        o_ref[...] = o - lse

    return pl.pallas_call(
        body,
        grid=(_G,),
        in_specs=[pl.BlockSpec((2, _BR, D), lambda i: (0, i, 0)),
                  pl.BlockSpec((_BR, D), lambda i: (i, 0)),
                  pl.BlockSpec((_BR, 1), lambda i: (i, 0)),
                  pl.BlockSpec((1, D), lambda i: (0, 0))],
        out_specs=pl.BlockSpec((_BR, D), lambda i: (i, 0)),
        out_shape=jax.ShapeDtypeStruct((_A, D), jnp.float32),
    )(agg, p, s, b)


def kernel(x, edge_index, W1, b1, W2, b2, W3, b3):
    src = edge_index[0].astype(jnp.int32)
    dst = edge_index[1].astype(jnp.int32)
    pad = _N + (jnp.arange(_EPAD - _E, dtype=jnp.int32) % (_A - _N))
    srcp = jnp.concatenate([src, pad]).reshape(_NW * _K, _C)
    dstp = jnp.concatenate([dst, pad]).reshape(_NW * _K, _C)

    x_pad = jnp.zeros((_A, x.shape[1]), x.dtype).at[:_N].set(x)
    zeros64 = jnp.zeros((_RPT, 64), jnp.float32)
    zeros16 = jnp.zeros((_RPT, 16), jnp.float32)
    ones16 = jnp.ones((_C, 16), jnp.float32)
    W3p = jnp.zeros((W3.shape[0], 16), jnp.float32).at[:, :2].set(W3)
    b1r = b1.reshape(1, -1)
    b2r = b2.reshape(1, -1)
    b3r = jnp.zeros((1, 16), jnp.float32).at[0, :2].set(b3)

    deg2 = _sc_deg(dstp, ones16, zeros16)
    p1, s, ps1 = _tc_prep(x_pad, W1, deg2)

    agg1 = _sc_segsum(64, ps1, srcp, dstp, zeros64)
    p2, ps2 = _tc_layer(agg1, p1, s, b1r, W2)

    agg2 = _sc_segsum(64, ps2, srcp, dstp, zeros64)
    p3, ps3 = _tc_layer(agg2, p2, s, b2r, W3p)

    agg3 = _sc_segsum(16, ps3, srcp, dstp, zeros16)
    out16 = _tc_final(agg3, p3, s, b3r)

    return out16[:_N, :2]
